# Initial kernel scaffold; baseline (speedup 1.0000x reference)
import jax
import jax.numpy as jnp
from jax.experimental import pallas as pl


def kernel(inputs, table):
    raise NotImplementedError("write your pallas kernel here")


# speedup vs baseline: 2.1627x; 2.1627x over previous
import functools

import jax
import jax.numpy as jnp
from jax import lax
from jax.experimental import pallas as pl
from jax.experimental.pallas import tpu as pltpu
from jax.experimental.pallas import tpu_sc as plsc

BSZ = 4096
MSL = 200
EMBDIM = 32

NC = 2
NS = 16
NW = NC * NS
B_PER_W = BSZ // NW
IDX_W = 100
ROWS_PER_B = MSL // IDX_W
CB = 4
G = CB * ROWS_PER_B
NCHUNK = B_PER_W // CB


def _body(idx_hbm, table_hbm, out_hbm, idx_v, rows_v, out_v, sem):
    wid = lax.axis_index("s") * NC + lax.axis_index("c")
    row2_base = wid * (B_PER_W * ROWS_PER_B)

    def chunk(c, carry):
        pltpu.sync_copy(idx_hbm.at[pl.ds(row2_base + c * G, G)], idx_v)
        handles = [
            pltpu.async_copy(table_hbm.at[idx_v.at[j]], rows_v.at[j], sem)
            for j in range(G)
        ]
        for h in handles:
            h.wait()
        zero = jnp.zeros((16,), jnp.float32)
        for b in range(CB):
            def red(j, accs):
                a0, a1 = accs
                a0 = a0 + rows_v[2 * b, j, 0:16]
                a1 = a1 + rows_v[2 * b, j, 16:32]
                a0 = a0 + rows_v[2 * b + 1, j, 0:16]
                a1 = a1 + rows_v[2 * b + 1, j, 16:32]
                return a0, a1
            a0, a1 = lax.fori_loop(0, IDX_W, red, (zero, zero))
            out_v[c * CB + b, 0:16] = a0
            out_v[c * CB + b, 16:32] = a1
        return carry

    lax.fori_loop(0, NCHUNK, chunk, 0)
    pltpu.sync_copy(out_v, out_hbm.at[pl.ds(wid * B_PER_W, B_PER_W)])


@jax.jit
def _emb_sum(idx2, table):
    mesh = plsc.VectorSubcoreMesh(
        core_axis_name="c", subcore_axis_name="s", num_cores=NC, num_subcores=NS
    )
    return pl.kernel(
        _body,
        out_type=jax.ShapeDtypeStruct((BSZ, EMBDIM), jnp.float32),
        mesh=mesh,
        scratch_types=[
            pltpu.VMEM((G, IDX_W), jnp.int32),
            pltpu.VMEM((G, IDX_W, EMBDIM), jnp.float32),
            pltpu.VMEM((B_PER_W, EMBDIM), jnp.float32),
            pltpu.SemaphoreType.DMA,
        ],
        compiler_params=pltpu.CompilerParams(use_tc_tiling_on_sc=False),
    )(idx2, table)


def kernel(inputs, table):
    idx2 = inputs.reshape(BSZ * ROWS_PER_B, IDX_W)
    return _emb_sum(idx2, table)


# speedup vs baseline: 2.4009x; 1.1101x over previous
import functools

import jax
import jax.numpy as jnp
from jax import lax
from jax.experimental import pallas as pl
from jax.experimental.pallas import tpu as pltpu
from jax.experimental.pallas import tpu_sc as plsc

BSZ = 4096
MSL = 200
EMBDIM = 32

NC = 2
NS = 16
NW = NC * NS
B_PER_W = BSZ // NW
IDX_W = 100
ROWS_PER_B = MSL // IDX_W
CB = 4
G = CB * ROWS_PER_B
NCHUNK = B_PER_W // CB


def _body(idx_hbm, table_hbm, out_hbm, idx_v, rows_v, out_v, sem0, sem1):
    wid = lax.axis_index("s") * NC + lax.axis_index("c")
    row2_base = wid * (B_PER_W * ROWS_PER_B)
    pltpu.sync_copy(idx_hbm.at[pl.ds(row2_base, B_PER_W * ROWS_PER_B)], idx_v)
    sems = (sem0, sem1)

    def fire(c, db):
        for j in range(G):
            pltpu.async_copy(
                table_hbm.at[idx_v.at[c * G + j]], rows_v.at[db, j], sems[db]
            )

    def drain(c, db):
        for j in range(G):
            pltpu.make_async_copy(
                table_hbm.at[idx_v.at[c * G + j]], rows_v.at[db, j], sems[db]
            ).wait()

    def reduce(c, db):
        zero = jnp.zeros((16,), jnp.float32)
        for b in range(CB):
            r0, r1 = 2 * b, 2 * b + 1

            @plsc.parallel_loop(0, IDX_W, unroll=4, carry=(zero, zero, zero, zero))
            def accs(j, carry):
                c00, c01, c10, c11 = carry
                c00 = c00 + rows_v[db, r0, j, 0:16]
                c01 = c01 + rows_v[db, r0, j, 16:32]
                c10 = c10 + rows_v[db, r1, j, 0:16]
                c11 = c11 + rows_v[db, r1, j, 16:32]
                return c00, c01, c10, c11

            c00, c01, c10, c11 = accs
            out_v[c * CB + b, 0:16] = c00 + c10
            out_v[c * CB + b, 16:32] = c01 + c11

    fire(0, 0)
    fire(1, 1)

    def step(i, carry):
        c0 = 2 * i
        drain(c0, 0)
        reduce(c0, 0)
        fire(c0 + 2, 0)
        drain(c0 + 1, 1)
        reduce(c0 + 1, 1)
        fire(c0 + 3, 1)
        return carry

    lax.fori_loop(0, NCHUNK // 2 - 1, step, 0)
    drain(NCHUNK - 2, 0)
    reduce(NCHUNK - 2, 0)
    drain(NCHUNK - 1, 1)
    reduce(NCHUNK - 1, 1)
    pltpu.sync_copy(out_v, out_hbm.at[pl.ds(wid * B_PER_W, B_PER_W)])


@jax.jit
def _emb_sum(idx2, table):
    mesh = plsc.VectorSubcoreMesh(
        core_axis_name="c", subcore_axis_name="s", num_cores=NC, num_subcores=NS
    )
    return pl.kernel(
        _body,
        out_type=jax.ShapeDtypeStruct((BSZ, EMBDIM), jnp.float32),
        mesh=mesh,
        scratch_types=[
            pltpu.VMEM((B_PER_W * ROWS_PER_B, IDX_W), jnp.int32),
            pltpu.VMEM((2, G, IDX_W, EMBDIM), jnp.float32),
            pltpu.VMEM((B_PER_W, EMBDIM), jnp.float32),
            pltpu.SemaphoreType.DMA,
            pltpu.SemaphoreType.DMA,
        ],
        compiler_params=pltpu.CompilerParams(use_tc_tiling_on_sc=False),
    )(idx2, table)


def kernel(inputs, table):
    idx2 = inputs.reshape(BSZ * ROWS_PER_B, IDX_W)
    return _emb_sum(idx2, table)


# speedup vs baseline: 2.4056x; 1.0020x over previous
import jax
import jax.numpy as jnp
from jax import lax
from jax.experimental import pallas as pl
from jax.experimental.pallas import tpu as pltpu
from jax.experimental.pallas import tpu_sc as plsc

BSZ = 4096
MSL = 200
EMBDIM = 32

NC = 2
NS = 16
NW = NC * NS
B_PER_W = BSZ // NW
CB = 4
NCHUNK = B_PER_W // CB
HALF = MSL // 2


def _body(idx_hbm, table_hbm, out_hbm, idx_v, rows_v, out_v, sem0, sem1):
    wid = lax.axis_index("s") * NC + lax.axis_index("c")
    b_base = wid * B_PER_W
    pltpu.sync_copy(idx_hbm.at[pl.ds(b_base, B_PER_W)], idx_v)
    sems = (sem0, sem1)

    def fire(c, db):
        for b in range(CB):
            pltpu.async_copy(
                table_hbm.at[idx_v.at[c * CB + b]], rows_v.at[db, b], sems[db]
            )

    def drain(c, db):
        for b in range(CB):
            pltpu.make_async_copy(
                table_hbm.at[idx_v.at[c * CB + b]], rows_v.at[db, b], sems[db]
            ).wait()

    def reduce(c, db):
        zero = jnp.zeros((16,), jnp.float32)
        for b in range(CB):

            @plsc.parallel_loop(0, HALF, unroll=4, carry=(zero, zero, zero, zero))
            def accs(j, carry):
                c00, c01, c10, c11 = carry
                c00 = c00 + rows_v[db, b, j, 0:16]
                c01 = c01 + rows_v[db, b, j, 16:32]
                c10 = c10 + rows_v[db, b, j + HALF, 0:16]
                c11 = c11 + rows_v[db, b, j + HALF, 16:32]
                return c00, c01, c10, c11

            c00, c01, c10, c11 = accs
            out_v[c * CB + b, 0:16] = c00 + c10
            out_v[c * CB + b, 16:32] = c01 + c11

    fire(0, 0)
    fire(1, 1)

    def step(i, carry):
        c0 = 2 * i
        drain(c0, 0)
        reduce(c0, 0)
        fire(c0 + 2, 0)
        drain(c0 + 1, 1)
        reduce(c0 + 1, 1)
        fire(c0 + 3, 1)
        return carry

    lax.fori_loop(0, NCHUNK // 2 - 1, step, 0)
    drain(NCHUNK - 2, 0)
    reduce(NCHUNK - 2, 0)
    drain(NCHUNK - 1, 1)
    reduce(NCHUNK - 1, 1)
    pltpu.sync_copy(out_v, out_hbm.at[pl.ds(b_base, B_PER_W)])


@jax.jit
def _emb_sum(idx, table):
    mesh = plsc.VectorSubcoreMesh(
        core_axis_name="c", subcore_axis_name="s", num_cores=NC, num_subcores=NS
    )
    return pl.kernel(
        _body,
        out_type=jax.ShapeDtypeStruct((BSZ, EMBDIM), jnp.float32),
        mesh=mesh,
        scratch_types=[
            pltpu.VMEM((B_PER_W, MSL), jnp.int32),
            pltpu.VMEM((2, CB, MSL, EMBDIM), jnp.float32),
            pltpu.VMEM((B_PER_W, EMBDIM), jnp.float32),
            pltpu.SemaphoreType.DMA,
            pltpu.SemaphoreType.DMA,
        ],
        compiler_params=pltpu.CompilerParams(use_tc_tiling_on_sc=False),
    )(idx, table)


def kernel(inputs, table):
    return _emb_sum(inputs, table)
